# speedup vs baseline: 2.8167x; 2.8167x over previous
import functools

import jax
import jax.numpy as jnp
from jax import lax
from jax.experimental import pallas as pl
from jax.experimental.pallas import tpu as pltpu
from jax.experimental.pallas import tpu_sc as plsc

_CHUNK = 128


@functools.cache
def _build(B, C, D_mu, D_al):
    info = plsc.get_sparse_core_info()
    NC, NS = info.num_cores, info.num_subcores
    NW = NC * NS
    b_per_w = B // NW
    assert B % (NW * _CHUNK) == 0
    n_chunks = b_per_w // _CHUNK

    mesh = plsc.VectorSubcoreMesh(core_axis_name="c", subcore_axis_name="s")

    @functools.partial(
        pl.kernel,
        mesh=mesh,
        out_type=[
            jax.ShapeDtypeStruct((B, D_mu), jnp.float32),
            jax.ShapeDtypeStruct((B, D_mu), jnp.float32),
            jax.ShapeDtypeStruct((B, D_al), jnp.float32),
        ],
        scratch_types=[
            pltpu.VMEM((n_chunks, _CHUNK), jnp.int32),
            pltpu.VMEM((_CHUNK, D_mu), jnp.float32),
            pltpu.VMEM((_CHUNK, D_mu), jnp.float32),
            pltpu.VMEM((_CHUNK, D_al), jnp.float32),
            pltpu.SemaphoreType.DMA,
        ],
    )
    def gather_kernel(ids_hbm, mu_hbm, sg_hbm, al_hbm,
                      mu_out, sg_out, al_out,
                      idx_v, mu_v, sg_v, al_v, sem):
        wid = lax.axis_index("s") * NC + lax.axis_index("c")
        pltpu.sync_copy(ids_hbm.at[pl.ds(wid * n_chunks, n_chunks)], idx_v)
        for j in range(n_chunks):
            row_ids = idx_v.at[j]
            cp_mu = pltpu.async_copy(mu_hbm.at[row_ids], mu_v, sem)
            cp_sg = pltpu.async_copy(sg_hbm.at[row_ids], sg_v, sem)
            cp_al = pltpu.async_copy(al_hbm.at[row_ids], al_v, sem)
            cp_mu.wait()
            cp_sg.wait()
            cp_al.wait()
            base = (wid * n_chunks + j) * _CHUNK
            pltpu.sync_copy(mu_v, mu_out.at[pl.ds(base, _CHUNK)])
            pltpu.sync_copy(sg_v, sg_out.at[pl.ds(base, _CHUNK)])
            pltpu.sync_copy(al_v, al_out.at[pl.ds(base, _CHUNK)])

    return gather_kernel


def kernel(category_ids, mu, sigma, alpha):
    B = category_ids.shape[0]
    C, K, _ = mu.shape
    D_mu = K * 3
    D_al = K * alpha.shape[2]
    ids2 = category_ids.astype(jnp.int32).reshape(B // _CHUNK, _CHUNK)
    mu2 = mu.reshape(C, D_mu)
    sg2 = sigma.reshape(C, D_mu)
    al2 = alpha.reshape(C, D_al)
    f = _build(B, C, D_mu, D_al)
    mu_o, sg_o, al_o = f(ids2, mu2, sg2, al2)
    return (mu_o.reshape(B, K, 3), sg_o.reshape(B, K, 3),
            al_o.reshape(B, K, alpha.shape[2]))
